# baseline (speedup 1.0000x reference)
import functools

import jax
import jax.numpy as jnp
from jax import lax
from jax.experimental import pallas as pl
from jax.experimental.pallas import tpu as pltpu
from jax.experimental.pallas import tpu_sc as plsc

NUM_NUMERICAL = 13
NUM_EMBED = 26
VOCAB = 100000
D = 64
BATCH = 16384
OUT_W = NUM_NUMERICAL + NUM_EMBED * D

NC = 2
NS = 16
NW = NC * NS

IDX_TOTAL = BATCH * NUM_EMBED
IDX_PER_W = IDX_TOTAL // NW
CHUNK = 128
GATHERS_PER_STEP = 4
STEP = CHUNK * GATHERS_PER_STEP
STEPS = IDX_PER_W // STEP

_VB = 1024
_VBLKS = (VOCAB + _VB - 1) // _VB


def _relayout_body(t_ref, o_ref):
    t = t_ref[0].T
    o_ref[0, :, 0:D] = t
    o_ref[0, :, D:2 * D] = t


_relayout = pl.pallas_call(
    _relayout_body,
    out_shape=jax.ShapeDtypeStruct((NUM_EMBED, VOCAB, 2 * D), jnp.float32),
    grid=(NUM_EMBED, _VBLKS),
    in_specs=[pl.BlockSpec((1, D, _VB), lambda j, k: (j, 0, k))],
    out_specs=pl.BlockSpec((1, _VB, 2 * D), lambda j, k: (j, k, 0)),
)

_mesh = plsc.VectorSubcoreMesh(core_axis_name="c", subcore_axis_name="s")


@functools.partial(
    pl.kernel,
    out_type=jax.ShapeDtypeStruct((IDX_TOTAL, 2 * D), jnp.float32),
    mesh=_mesh,
    scratch_types=[
        pltpu.VMEM((IDX_PER_W,), jnp.int32),
        pltpu.VMEM((STEP, 2 * D), jnp.float32),
        pltpu.SemaphoreType.DMA,
    ],
)
def _sc_gather(tables_hbm, idx_hbm, out_hbm, idx_v, buf_v, sem):
    wid = lax.axis_index("s") * NC + lax.axis_index("c")
    base = wid * IDX_PER_W
    pltpu.sync_copy(idx_hbm.at[pl.ds(base, IDX_PER_W)], idx_v)

    @pl.loop(0, STEPS)
    def _(step):
        off = step * STEP
        copies = []
        for g in range(GATHERS_PER_STEP):
            copies.append(
                pltpu.async_copy(
                    tables_hbm.at[idx_v.at[pl.ds(off + g * CHUNK, CHUNK)]],
                    buf_v.at[pl.ds(g * CHUNK, CHUNK)],
                    sem,
                )
            )
        for c in copies:
            c.wait()
        pltpu.sync_copy(buf_v, out_hbm.at[pl.ds(base + off, STEP)])


_RB = 256


def _concat_body(x_ref, emb_ref, o_ref):
    o_ref[:, :NUM_NUMERICAL] = x_ref[:, :NUM_NUMERICAL]
    for j in range(NUM_EMBED):
        col = NUM_NUMERICAL + j * D
        o_ref[:, col:col + D] = emb_ref[j, :, 0:D]


_concat = pl.pallas_call(
    _concat_body,
    out_shape=jax.ShapeDtypeStruct((BATCH, OUT_W), jnp.float32),
    grid=(BATCH // _RB,),
    in_specs=[
        pl.BlockSpec((_RB, NUM_NUMERICAL + NUM_EMBED), lambda i: (i, 0)),
        pl.BlockSpec((NUM_EMBED, _RB, 2 * D), lambda i: (0, i, 0)),
    ],
    out_specs=pl.BlockSpec((_RB, OUT_W), lambda i: (i, 0)),
)


def kernel(x, tables):
    tables_cm = jnp.swapaxes(tables, 1, 2)
    trows = _relayout(tables_cm).reshape(NUM_EMBED * VOCAB, 2 * D)
    idx = (x[:, NUM_NUMERICAL:].astype(jnp.int32).T
           + (jnp.arange(NUM_EMBED, dtype=jnp.int32) * VOCAB)[:, None])
    emb = _sc_gather(trows, idx.reshape(-1))
    return _concat(x, emb.reshape(NUM_EMBED, BATCH, 2 * D))
